# baseline (speedup 1.0000x reference)
import functools
import jax
import jax.numpy as jnp
from jax import lax
from jax.experimental import pallas as pl
from jax.experimental.pallas import tpu as pltpu
from jax.experimental.pallas import tpu_sc as plsc

D_MODEL = 64
SCALE = 8.0
LANES = 16
CHUNK = 512


def kernel(lut, x):
    batch_shape = x.shape
    xf = x.reshape(-1).astype(jnp.int32)
    total = xf.shape[0]

    info = plsc.get_sparse_core_info()
    num_workers = info.num_cores * info.num_subcores
    per_worker = total // num_workers
    n_chunks = per_worker // CHUNK
    num_cores = info.num_cores

    mesh = plsc.VectorSubcoreMesh(core_axis_name="c", subcore_axis_name="s")

    @functools.partial(
        pl.kernel,
        mesh=mesh,
        out_type=jax.ShapeDtypeStruct((total, D_MODEL), jnp.float32),
        scratch_types=[
            pltpu.VMEM((per_worker,), jnp.int32),
            pltpu.VMEM((CHUNK, D_MODEL), jnp.float32),
            pltpu.SemaphoreType.DMA,
        ],
        compiler_params=pltpu.CompilerParams(use_tc_tiling_on_sc=False),
    )
    def gather_scale(lut_hbm, x_hbm, out_hbm, idx_v, buf, sem):
        wid = lax.axis_index("s") * num_cores + lax.axis_index("c")
        base = wid * per_worker
        pltpu.sync_copy(x_hbm.at[pl.ds(base, per_worker)], idx_v)

        def chunk_body(c, carry):
            pltpu.async_copy(
                lut_hbm.at[idx_v.at[pl.ds(c * CHUNK, CHUNK)]], buf, sem
            ).wait()

            def row_body(r, carry2):
                for j in range(D_MODEL // LANES):
                    buf[r, pl.ds(j * LANES, LANES)] = (
                        buf[r, pl.ds(j * LANES, LANES)] * SCALE
                    )
                return carry2

            lax.fori_loop(0, CHUNK, row_body, 0)
            pltpu.sync_copy(buf, out_hbm.at[pl.ds(base + c * CHUNK, CHUNK)])
            return carry

        lax.fori_loop(0, n_chunks, chunk_body, 0)

    out = gather_scale(lut, xf)
    return out.reshape(batch_shape + (D_MODEL,))


# speedup vs baseline: 1.1202x; 1.1202x over previous
import functools
import jax
import jax.numpy as jnp
from jax import lax
from jax.experimental import pallas as pl
from jax.experimental.pallas import tpu as pltpu
from jax.experimental.pallas import tpu_sc as plsc

D_MODEL = 64
SCALE = 8.0
LANES = 16
CHUNK = 320
NBUF = 4


def kernel(lut, x):
    batch_shape = x.shape
    xf = x.reshape(-1).astype(jnp.int32)
    total = xf.shape[0]

    info = plsc.get_sparse_core_info()
    num_workers = info.num_cores * info.num_subcores
    per_worker = total // num_workers
    n_chunks = per_worker // CHUNK
    num_cores = info.num_cores

    mesh = plsc.VectorSubcoreMesh(core_axis_name="c", subcore_axis_name="s")

    @functools.partial(
        pl.kernel,
        mesh=mesh,
        out_type=jax.ShapeDtypeStruct((total, D_MODEL), jnp.float32),
        scratch_types=[
            pltpu.VMEM((per_worker,), jnp.int32),
            [pltpu.VMEM((CHUNK, D_MODEL), jnp.float32) for _ in range(NBUF)],
            [pltpu.SemaphoreType.DMA for _ in range(NBUF)],
            [pltpu.SemaphoreType.DMA for _ in range(NBUF)],
        ],
        compiler_params=pltpu.CompilerParams(use_tc_tiling_on_sc=False),
    )
    def gather_scale(lut_hbm, x_hbm, out_hbm, idx_v, bufs, gsems, wsems):
        wid = lax.axis_index("s") * num_cores + lax.axis_index("c")
        base = wid * per_worker
        pltpu.sync_copy(x_hbm.at[pl.ds(base, per_worker)], idx_v)

        def gather_start(j, b):
            pltpu.async_copy(
                lut_hbm.at[idx_v.at[pl.ds(j * CHUNK, CHUNK)]], bufs[b], gsems[b]
            )

        def gather_wait(j, b):
            pltpu.make_async_copy(
                lut_hbm.at[idx_v.at[pl.ds(j * CHUNK, CHUNK)]], bufs[b], gsems[b]
            ).wait()

        def write_start(j, b):
            pltpu.async_copy(
                bufs[b], out_hbm.at[pl.ds(base + j * CHUNK, CHUNK)], wsems[b]
            )

        def write_wait(j, b):
            pltpu.make_async_copy(
                bufs[b], out_hbm.at[pl.ds(base + j * CHUNK, CHUNK)], wsems[b]
            ).wait()

        def scale(b):
            buf = bufs[b]

            def row_body(r2, carry):
                r = r2 * 2
                for u in range(2):
                    for q in range(D_MODEL // LANES):
                        buf[r + u, pl.ds(q * LANES, LANES)] = (
                            buf[r + u, pl.ds(q * LANES, LANES)] * SCALE
                        )
                return carry

            lax.fori_loop(0, CHUNK // 2, row_body, 0)

        gather_start(0, 0)
        for j in range(NBUF - 1):
            gather_start(j + 1, j + 1)
            gather_wait(j, j)
            scale(j)
            write_start(j, j)

        def steady(c, carry):
            for b in range(NBUF):
                j = (NBUF - 1) + c * NBUF + b
                bf = (NBUF - 1 + b) % NBUF
                write_wait(j - (NBUF - 1), b)
                gather_start(j + 1, b)
                gather_wait(j, bf)
                scale(bf)
                write_start(j, bf)
            return carry

        lax.fori_loop(0, (n_chunks - NBUF) // NBUF, steady, 0)

        jl = n_chunks - 1
        bl = jl % NBUF
        gather_wait(jl, bl)
        scale(bl)
        write_start(jl, bl)
        for j in range(n_chunks - NBUF, n_chunks):
            write_wait(j, j % NBUF)

    out = gather_scale(lut, xf)
    return out.reshape(batch_shape + (D_MODEL,))
